# baseline (speedup 1.0000x reference)
import functools

import jax
import jax.numpy as jnp
from jax import lax
from jax.experimental import pallas as pl
from jax.experimental.pallas import tpu as pltpu
from jax.experimental.pallas import tpu_sc as plsc

_NC = 2
_NS = 16
_NW = _NC * _NS
_L = 16

_BATCH = 16384
_D = 64
_BPW = _BATCH // _NW
_CHUNK = 256
_NCHUNK = _BPW // _CHUNK
_GROUPS = _CHUNK // _L


def _sc_body(u_hbm, i_hbm, mu_hbm, bu_hbm, bi_hbm, p_hbm, q_hbm, out_hbm,
             uidx_v, iidx_v, u2_v, i2_v, pu_v, qi_v, bu_v, bi_v, mu_v,
             out_v, sem):
    wid = lax.axis_index("s") * _NC + lax.axis_index("c")
    base = wid * _BPW
    pltpu.sync_copy(mu_hbm, mu_v)
    mu_vec = mu_v[...]

    for chunk in range(_NCHUNK):
        cbase = base + chunk * _CHUNK
        pltpu.sync_copy(u_hbm.at[pl.ds(cbase, _CHUNK)], uidx_v)
        pltpu.sync_copy(i_hbm.at[pl.ds(cbase, _CHUNK)], iidx_v)
        for k in range(_CHUNK // _L):
            sl = pl.ds(k * _L, _L)
            u2_v[sl] = lax.shift_right_logical(uidx_v[sl], 1)
            i2_v[sl] = lax.shift_right_logical(iidx_v[sl], 1)
        cps = [
            pltpu.async_copy(p_hbm.at[u2_v], pu_v, sem),
            pltpu.async_copy(q_hbm.at[i2_v], qi_v, sem),
            pltpu.async_copy(bu_hbm.at[uidx_v], bu_v, sem),
            pltpu.async_copy(bi_hbm.at[iidx_v], bi_v, sem),
        ]
        for cp in cps:
            cp.wait()

        lane_iota = lax.iota(jnp.int32, _L)
        lane_masks = [lane_iota == r for r in range(_L)]

        def group(g, carry):
            gbase = pl.multiple_of(g * _L, _L)
            d_ee = jnp.zeros((_L,), jnp.float32)
            d_eo = jnp.zeros((_L,), jnp.float32)
            d_oe = jnp.zeros((_L,), jnp.float32)
            d_oo = jnp.zeros((_L,), jnp.float32)
            for r in range(_L):
                b = gbase + r
                a_ee = jnp.zeros((_L,), jnp.float32)
                a_eo = jnp.zeros((_L,), jnp.float32)
                a_oe = jnp.zeros((_L,), jnp.float32)
                a_oo = jnp.zeros((_L,), jnp.float32)
                for c in range(_D // _L):
                    pe = pu_v[b, pl.ds(c * _L, _L)]
                    po = pu_v[b, pl.ds(_D + c * _L, _L)]
                    qe = qi_v[b, pl.ds(c * _L, _L)]
                    qo = qi_v[b, pl.ds(_D + c * _L, _L)]
                    a_ee = a_ee + pe * qe
                    a_eo = a_eo + pe * qo
                    a_oe = a_oe + po * qe
                    a_oo = a_oo + po * qo
                m = lane_masks[r]
                d_ee = jnp.where(m, jnp.sum(a_ee), d_ee)
                d_eo = jnp.where(m, jnp.sum(a_eo), d_eo)
                d_oe = jnp.where(m, jnp.sum(a_oe), d_oe)
                d_oo = jnp.where(m, jnp.sum(a_oo), d_oo)
            sl = pl.ds(gbase, _L)
            u_even = (uidx_v[sl] & 1) == 0
            i_even = (iidx_v[sl] & 1) == 0
            dots = jnp.where(
                u_even,
                jnp.where(i_even, d_ee, d_eo),
                jnp.where(i_even, d_oe, d_oo))
            out_v[sl] = mu_vec + bu_v[sl] + bi_v[sl] + dots
            return carry

        lax.fori_loop(0, _GROUPS, group, 0)
        pltpu.sync_copy(out_v, out_hbm.at[pl.ds(cbase, _CHUNK)])


def kernel(u_idx, i_idx, mu, b_u, b_i, P, Q):
    u_idx = u_idx.astype(jnp.int32)
    i_idx = i_idx.astype(jnp.int32)
    mu_vec = jnp.broadcast_to(mu.astype(jnp.float32), (_L,))
    mesh = plsc.VectorSubcoreMesh(core_axis_name="c", subcore_axis_name="s")
    run = functools.partial(
        pl.kernel,
        mesh=mesh,
        compiler_params=pltpu.CompilerParams(
            needs_layout_passes=False, use_tc_tiling_on_sc=False),
        out_type=jax.ShapeDtypeStruct((_BATCH,), jnp.float32),
        scratch_types=[
            pltpu.VMEM((_CHUNK,), jnp.int32),
            pltpu.VMEM((_CHUNK,), jnp.int32),
            pltpu.VMEM((_CHUNK,), jnp.int32),
            pltpu.VMEM((_CHUNK,), jnp.int32),
            pltpu.VMEM((_CHUNK, 2 * _D), jnp.float32),
            pltpu.VMEM((_CHUNK, 2 * _D), jnp.float32),
            pltpu.VMEM((_CHUNK,), jnp.float32),
            pltpu.VMEM((_CHUNK,), jnp.float32),
            pltpu.VMEM((_L,), jnp.float32),
            pltpu.VMEM((_CHUNK,), jnp.float32),
            pltpu.SemaphoreType.DMA,
        ],
    )(_sc_body)
    return run(u_idx, i_idx, mu_vec, b_u, b_i,
               P.reshape(500000, 2 * _D), Q.reshape(500000, 2 * _D))
